# baseline (speedup 1.0000x reference)
import functools

import jax
import jax.numpy as jnp
from jax import lax
from jax.experimental import pallas as pl
from jax.experimental.pallas import tpu as pltpu
from jax.experimental.pallas import tpu_sc as plsc

_NC, _NS = 2, 16
_NW = _NC * _NS
_CHUNK = 128
_NBUF = 4


def _gather_body(idx_hbm, emb_hbm, out_hbm, idx_v, rows_v, gsem, wsem,
                 *, rows_per_worker, d):
    wid = lax.axis_index("s") * _NC + lax.axis_index("c")
    base = wid * rows_per_worker
    n_chunks = rows_per_worker // _CHUNK
    pchunk = _CHUNK // 2
    pltpu.sync_copy(idx_hbm.at[pl.ds(base, rows_per_worker)], idx_v)

    def outer(g):
        c0 = g * _NBUF
        descs = []
        for k in range(_NBUF):
            idx_slice = idx_v.at[pl.ds((c0 + k) * _CHUNK, _CHUNK)]
            descs.append(
                pltpu.async_copy(emb_hbm.at[idx_slice], rows_v.at[k], gsem))
        wdescs = []
        for k in range(_NBUF):
            descs[k].wait()
            dst = out_hbm.at[pl.ds(base + (c0 + k) * _CHUNK, _CHUNK)]
            wdescs.append(pltpu.async_copy(rows_v.at[k], dst, wsem))
        for k in range(_NBUF):
            wdescs[k].wait()

    pl.loop(0, n_chunks // _NBUF)(outer)


def _sc_gather(idxs_flat, emb):
    rows = idxs_flat.shape[0]
    d = emb.shape[1]
    rows_per_worker = rows // _NW
    mesh = plsc.VectorSubcoreMesh(core_axis_name="c", subcore_axis_name="s")
    body = functools.partial(_gather_body, rows_per_worker=rows_per_worker,
                             d=d)
    return pl.kernel(
        body,
        out_type=jax.ShapeDtypeStruct((rows, d), jnp.float32),
        mesh=mesh,
        scratch_types=[
            pltpu.VMEM((rows_per_worker,), jnp.int32),
            pltpu.VMEM((_NBUF, _CHUNK, d), jnp.float32),
            pltpu.SemaphoreType.DMA,
            pltpu.SemaphoreType.DMA,
        ],
        compiler_params=pltpu.CompilerParams(use_tc_tiling_on_sc=False),
    )(idxs_flat, emb)


def _mm_body(x2_ref, w2_ref, b_ref, *rest, d, bb, hist, has_alias):
    o_ref = rest[-1]
    y2 = jnp.dot(x2_ref[...], w2_ref[...], preferred_element_type=jnp.float32)
    y2 = y2 + b_ref[...]
    o_ref[...] = y2.reshape(bb, hist, o_ref.shape[2])


def _tc_matmul_half(x2, w2, b2, y_in, batch, hist, bb, boff):
    d2, dout2 = w2.shape
    dout = dout2 // 2
    pblk = bb * hist // 2
    nblk = x2.shape[0] // pblk
    off_blocks = boff // bb
    kwargs = {}
    operands = (x2, w2, b2)
    in_specs = [
        pl.BlockSpec((pblk, d2), lambda i: (i, 0)),
        pl.BlockSpec((d2, dout2), lambda i: (0, 0)),
        pl.BlockSpec((1, dout2), lambda i: (0, 0)),
    ]
    if y_in is not None:
        kwargs["input_output_aliases"] = {3: 0}
        operands = operands + (y_in,)
        in_specs.append(pl.BlockSpec(
            (bb, hist, dout), lambda i: (i + off_blocks, 0, 0)))
    return pl.pallas_call(
        functools.partial(_mm_body, d=d2 // 2, bb=bb, hist=hist,
                          has_alias=y_in is not None),
        grid=(nblk,),
        in_specs=in_specs,
        out_specs=pl.BlockSpec((bb, hist, dout),
                               lambda i: (i + off_blocks, 0, 0)),
        out_shape=jax.ShapeDtypeStruct((batch, hist, dout), jnp.float32),
        **kwargs,
    )(*operands)


def kernel(idxs, emb, W, b):
    batch, hist = idxs.shape
    rows = batch * hist
    d = emb.shape[1]
    idxs_flat = idxs.reshape(rows).astype(jnp.int32)
    z = jnp.zeros_like(W)
    w2 = jnp.concatenate(
        [jnp.concatenate([W, z], axis=1), jnp.concatenate([z, W], axis=1)],
        axis=0)
    b2d = b.reshape(1, -1)
    b2 = jnp.concatenate([b2d, b2d], axis=1)
    half = rows // 2
    xa = _sc_gather(jax.lax.slice(idxs_flat, (0,), (half,)), emb)
    xb = _sc_gather(jax.lax.slice(idxs_flat, (half,), (rows,)), emb)
    x2a = xa.reshape(half // 2, 2 * d)
    x2b = xb.reshape(half // 2, 2 * d)
    ya = _tc_matmul_half(x2a, w2, b2, None, batch, hist, 256, 0)
    return _tc_matmul_half(x2b, w2, b2, ya, batch, hist, 256, batch // 2)
